# baseline (speedup 1.0000x reference)
import functools

import jax
import jax.numpy as jnp
from jax import lax
from jax.experimental import pallas as pl
from jax.experimental.pallas import tpu as pltpu
from jax.experimental.pallas import tpu_sc as plsc

_N_LIT = 10000
_N_CLS = 5000
_E = 320000
_C = 128
_OUT_DIM = 2

_NW = 32
_EPW = _E // _NW
_K = 128
_EPW_PAD = 10240
_NCHUNK = _EPW_PAD // _K
_N_CLS_PAD = 5120
_N_LIT_PAD = 10112

_MESH = plsc.VectorSubcoreMesh(core_axis_name="c", subcore_axis_name="s")


def _make_seg_sum(n_pad):
    stripe = n_pad // 16

    @functools.partial(
        pl.kernel,
        out_type=jax.ShapeDtypeStruct((2, n_pad, _C), jnp.float32),
        mesh=_MESH,
        scratch_types=[
            pltpu.VMEM((_NCHUNK, _K), jnp.int32),
            pltpu.VMEM((_NCHUNK, _K), jnp.int32),
            pltpu.VMEM((_K, _C), jnp.float32),
            pltpu.MemorySpace.VMEM_SHARED((n_pad, _C), jnp.float32),
            pltpu.SemaphoreType.DMA,
        ],
    )
    def seg_sum(table, esrc, edst, zeros, out,
                idx_s, idx_d, rows, acc, sem):
        c = lax.axis_index("c")
        s = lax.axis_index("s")
        wid = s * 2 + c
        pltpu.sync_copy(zeros.at[pl.ds(s * stripe, stripe)],
                        acc.at[pl.ds(s * stripe, stripe)])
        pltpu.sync_copy(esrc.at[wid], idx_s)
        pltpu.sync_copy(edst.at[wid], idx_d)
        plsc.subcore_barrier()

        def body(j, carry):
            pltpu.async_copy(table.at[idx_s.at[j]], rows, sem).wait()
            pltpu.sync_copy(rows, acc.at[idx_d.at[j]], add=True)
            return carry

        lax.fori_loop(0, _NCHUNK, body, 0)
        plsc.subcore_barrier()
        pltpu.sync_copy(acc.at[pl.ds(s * stripe, stripe)],
                        out.at[c, pl.ds(s * stripe, stripe)])

    return seg_sum


_seg_to_cls = _make_seg_sum(_N_CLS_PAD)
_seg_to_lit = _make_seg_sum(_N_LIT_PAD)

def _make_count(n_pad):
    stripe = n_pad // 16

    @functools.partial(
        pl.kernel,
        out_type=jax.ShapeDtypeStruct((2, n_pad, _C), jnp.float32),
        mesh=_MESH,
        scratch_types=[
            pltpu.VMEM((_NCHUNK, _K), jnp.int32),
            pltpu.VMEM((_K, _C), jnp.float32),
            pltpu.MemorySpace.VMEM_SHARED((n_pad, _C), jnp.float32),
        ],
    )
    def count(edst, zeros, ones, out, idx_d, ones_v, acc):
        c = lax.axis_index("c")
        s = lax.axis_index("s")
        wid = s * 2 + c
        pltpu.sync_copy(zeros.at[pl.ds(s * stripe, stripe)],
                        acc.at[pl.ds(s * stripe, stripe)])
        pltpu.sync_copy(ones, ones_v)
        pltpu.sync_copy(edst.at[wid], idx_d)
        plsc.subcore_barrier()

        def body(j, carry):
            pltpu.sync_copy(ones_v, acc.at[idx_d.at[j]], add=True)
            return carry

        lax.fori_loop(0, _NCHUNK, body, 0)
        plsc.subcore_barrier()
        pltpu.sync_copy(acc.at[pl.ds(s * stripe, stripe)],
                        out.at[c, pl.ds(s * stripe, stripe)])

    return count


_cnt_cls_kernel = _make_count(_N_CLS_PAD)
_cnt_lit_kernel = _make_count(_N_LIT_PAD)


def _mlp2(x, w1, b1, w2, b2, blk):
    n, d1 = x.shape
    dh = w1.shape[1]
    do = w2.shape[1]

    def body(x_ref, w1_ref, b1_ref, w2_ref, b2_ref, o_ref):
        z = jnp.dot(x_ref[...], w1_ref[...],
                    preferred_element_type=jnp.float32) + b1_ref[...]
        h = z * jax.nn.sigmoid(z)
        o_ref[...] = jnp.dot(h, w2_ref[...],
                             preferred_element_type=jnp.float32) + b2_ref[...]

    return pl.pallas_call(
        body,
        grid=(n // blk,),
        in_specs=[
            pl.BlockSpec((blk, d1), lambda i: (i, 0)),
            pl.BlockSpec((d1, dh), lambda i: (0, 0)),
            pl.BlockSpec((1, dh), lambda i: (0, 0)),
            pl.BlockSpec((dh, do), lambda i: (0, 0)),
            pl.BlockSpec((1, do), lambda i: (0, 0)),
        ],
        out_specs=pl.BlockSpec((blk, do), lambda i: (i, 0)),
        out_shape=jax.ShapeDtypeStruct((n, do), jnp.float32),
    )(x, w1, b1.reshape(1, dh), w2, b2.reshape(1, do))


def _layer_update(h, partials, cnts, w, b, blk):
    n = h.shape[0]

    def body(h_ref, p_ref, c_ref, w_ref, b_ref, o_ref):
        hx = h_ref[...]
        cnt = c_ref[0] + c_ref[1]
        agg = (p_ref[0] + p_ref[1]) / jnp.maximum(cnt, 1.0)
        z = (jnp.dot(hx, w_ref[0], preferred_element_type=jnp.float32)
             + jnp.dot(agg, w_ref[1], preferred_element_type=jnp.float32)
             + b_ref[...])
        o_ref[...] = hx + z * jax.nn.sigmoid(z)

    return pl.pallas_call(
        body,
        grid=(n // blk,),
        in_specs=[
            pl.BlockSpec((blk, _C), lambda i: (i, 0)),
            pl.BlockSpec((2, blk, _C), lambda i: (0, i, 0)),
            pl.BlockSpec((2, blk, _C), lambda i: (0, i, 0)),
            pl.BlockSpec((2, _C, _C), lambda i: (0, 0, 0)),
            pl.BlockSpec((1, _C), lambda i: (0, 0)),
        ],
        out_specs=pl.BlockSpec((blk, _C), lambda i: (i, 0)),
        out_shape=jax.ShapeDtypeStruct((n, _C), jnp.float32),
    )(h, partials, cnts, w, b.reshape(1, _C))


def kernel(x_lit, x_cls, edge_lit, edge_cls, enc_W1, enc_b1, enc_W2, enc_b2,
           lit_W, lit_b, cls_W, cls_b, out_W1, out_b1, out_W2, out_b2):
    def _pad(e, pad_val):
        e2 = e.reshape(_NW, _EPW)
        fill = jnp.full((_NW, _EPW_PAD - _EPW), pad_val, jnp.int32)
        return jnp.concatenate([e2, fill], axis=1)

    src_lit = _pad(edge_lit, 0).reshape(_NW, _NCHUNK, _K)
    src_cls = _pad(edge_cls, 0).reshape(_NW, _NCHUNK, _K)
    dst_cls = _pad(edge_cls, _N_CLS).reshape(_NW, _NCHUNK, _K)
    dst_lit = _pad(edge_lit, _N_LIT).reshape(_NW, _NCHUNK, _K)
    zeros128 = jnp.zeros((_N_LIT_PAD, _C), jnp.float32)
    ones128 = jnp.ones((_K, _C), jnp.float32)

    cnt_cls = _cnt_cls_kernel(dst_cls, zeros128, ones128)
    cnt_lit = _cnt_lit_kernel(dst_lit, zeros128, ones128)

    x_all = jnp.concatenate([x_lit, x_cls], axis=0)
    h_all = _mlp2(x_all, enc_W1, enc_b1, enc_W2, enc_b2, blk=1000)
    h_lit, h_cls = h_all[:_N_LIT], h_all[_N_LIT:]

    n_layers = lit_W.shape[0]
    cls_W2 = cls_W.reshape(n_layers, 2, _C, _C)
    lit_W2 = lit_W.reshape(n_layers, 2, _C, _C)

    for l in range(n_layers):
        p_cls = _seg_to_cls(h_lit, src_lit, dst_cls, zeros128)
        h_cls = _layer_update(h_cls, p_cls, cnt_cls, cls_W2[l], cls_b[l], blk=1000)
        p_lit = _seg_to_lit(h_cls, src_cls, dst_lit, zeros128)
        h_lit = _layer_update(h_lit, p_lit, cnt_lit, lit_W2[l], lit_b[l], blk=1000)

    hv = h_lit.reshape(_N_CLS, 2 * _C)
    w2p = jnp.zeros((2 * _C, _C), jnp.float32).at[:, :_OUT_DIM].set(out_W2)
    b2p = jnp.zeros((_C,), jnp.float32).at[:_OUT_DIM].set(out_b2)
    y = _mlp2(hv, out_W1, out_b1, w2p, b2p, blk=1000)
    return y[:, :_OUT_DIM]


# speedup vs baseline: 2.0147x; 2.0147x over previous
import functools

import jax
import jax.numpy as jnp
from jax import lax
from jax.experimental import pallas as pl
from jax.experimental.pallas import tpu as pltpu
from jax.experimental.pallas import tpu_sc as plsc

_N_LIT = 10000
_N_CLS = 5000
_E = 320000
_C = 128
_OUT_DIM = 2

_NW = 32
_EPW = _E // _NW
_K = 80
_EPW_PAD = 10000
_NCHUNK = _EPW_PAD // _K
_N_CLS_PAD = 5120
_N_LIT_PAD = 10112

_MESH = plsc.VectorSubcoreMesh(core_axis_name="c", subcore_axis_name="s")


def _make_seg_sum(n_pad):
    stripe = n_pad // 16

    @functools.partial(
        pl.kernel,
        out_type=jax.ShapeDtypeStruct((2, n_pad, _C), jnp.float32),
        mesh=_MESH,
        scratch_types=[
            pltpu.VMEM((_NCHUNK, _K), jnp.int32),
            pltpu.VMEM((_NCHUNK, _K), jnp.int32),
            pltpu.VMEM((_K, _C), jnp.float32),
            pltpu.MemorySpace.VMEM_SHARED((n_pad, _C), jnp.float32),
            pltpu.SemaphoreType.DMA,
        ],
    )
    def seg_sum(table, esrc, edst, zeros, out,
                idx_s, idx_d, rows, acc, sem):
        c = lax.axis_index("c")
        s = lax.axis_index("s")
        wid = s * 2 + c
        pltpu.sync_copy(zeros.at[pl.ds(s * stripe, stripe)],
                        acc.at[pl.ds(s * stripe, stripe)])
        pltpu.sync_copy(esrc.at[wid], idx_s)
        pltpu.sync_copy(edst.at[wid], idx_d)
        plsc.subcore_barrier()

        def body(j, carry):
            pltpu.async_copy(table.at[idx_s.at[j]], rows, sem).wait()
            pltpu.sync_copy(rows, acc.at[idx_d.at[j]], add=True)
            return carry

        lax.fori_loop(0, _NCHUNK, body, 0)
        plsc.subcore_barrier()
        pltpu.sync_copy(acc.at[pl.ds(s * stripe, stripe)],
                        out.at[c, pl.ds(s * stripe, stripe)])

    return seg_sum


_seg_to_cls = _make_seg_sum(_N_CLS_PAD)
_seg_to_lit = _make_seg_sum(_N_LIT_PAD)

def _make_count(n_pad):
    stripe = n_pad // 16

    @functools.partial(
        pl.kernel,
        out_type=jax.ShapeDtypeStruct((2, n_pad, _C), jnp.float32),
        mesh=_MESH,
        scratch_types=[
            pltpu.VMEM((_NCHUNK, _K), jnp.int32),
            pltpu.VMEM((_K, _C), jnp.float32),
            pltpu.MemorySpace.VMEM_SHARED((n_pad, _C), jnp.float32),
        ],
    )
    def count(edst, zeros, ones, out, idx_d, ones_v, acc):
        c = lax.axis_index("c")
        s = lax.axis_index("s")
        wid = s * 2 + c
        pltpu.sync_copy(zeros.at[pl.ds(s * stripe, stripe)],
                        acc.at[pl.ds(s * stripe, stripe)])
        pltpu.sync_copy(ones, ones_v)
        pltpu.sync_copy(edst.at[wid], idx_d)
        plsc.subcore_barrier()

        def body(j, carry):
            pltpu.sync_copy(ones_v, acc.at[idx_d.at[j]], add=True)
            return carry

        lax.fori_loop(0, _NCHUNK, body, 0)
        plsc.subcore_barrier()
        pltpu.sync_copy(acc.at[pl.ds(s * stripe, stripe)],
                        out.at[c, pl.ds(s * stripe, stripe)])

    return count


_cnt_cls_kernel = _make_count(_N_CLS_PAD)
_cnt_lit_kernel = _make_count(_N_LIT_PAD)


def _mlp2(x, w1, b1, w2, b2, blk):
    n, d1 = x.shape
    dh = w1.shape[1]
    do = w2.shape[1]

    def body(x_ref, w1_ref, b1_ref, w2_ref, b2_ref, o_ref):
        z = jnp.dot(x_ref[...], w1_ref[...],
                    preferred_element_type=jnp.float32) + b1_ref[...]
        h = z * jax.nn.sigmoid(z)
        o_ref[...] = jnp.dot(h, w2_ref[...],
                             preferred_element_type=jnp.float32) + b2_ref[...]

    return pl.pallas_call(
        body,
        grid=(n // blk,),
        in_specs=[
            pl.BlockSpec((blk, d1), lambda i: (i, 0)),
            pl.BlockSpec((d1, dh), lambda i: (0, 0)),
            pl.BlockSpec((1, dh), lambda i: (0, 0)),
            pl.BlockSpec((dh, do), lambda i: (0, 0)),
            pl.BlockSpec((1, do), lambda i: (0, 0)),
        ],
        out_specs=pl.BlockSpec((blk, do), lambda i: (i, 0)),
        out_shape=jax.ShapeDtypeStruct((n, do), jnp.float32),
    )(x, w1, b1.reshape(1, dh), w2, b2.reshape(1, do))


def _layer_update(h, partials, cnts, w, b, blk):
    n = h.shape[0]

    def body(h_ref, p_ref, c_ref, w_ref, b_ref, o_ref):
        hx = h_ref[...]
        cnt = c_ref[0] + c_ref[1]
        agg = (p_ref[0] + p_ref[1]) / jnp.maximum(cnt, 1.0)
        z = (jnp.dot(hx, w_ref[0], preferred_element_type=jnp.float32)
             + jnp.dot(agg, w_ref[1], preferred_element_type=jnp.float32)
             + b_ref[...])
        o_ref[...] = hx + z * jax.nn.sigmoid(z)

    return pl.pallas_call(
        body,
        grid=(n // blk,),
        in_specs=[
            pl.BlockSpec((blk, _C), lambda i: (i, 0)),
            pl.BlockSpec((2, blk, _C), lambda i: (0, i, 0)),
            pl.BlockSpec((2, blk, _C), lambda i: (0, i, 0)),
            pl.BlockSpec((2, _C, _C), lambda i: (0, 0, 0)),
            pl.BlockSpec((1, _C), lambda i: (0, 0)),
        ],
        out_specs=pl.BlockSpec((blk, _C), lambda i: (i, 0)),
        out_shape=jax.ShapeDtypeStruct((n, _C), jnp.float32),
    )(h, partials, cnts, w, b.reshape(1, _C))


def kernel(x_lit, x_cls, edge_lit, edge_cls, enc_W1, enc_b1, enc_W2, enc_b2,
           lit_W, lit_b, cls_W, cls_b, out_W1, out_b1, out_W2, out_b2):
    def _pad(e, pad_val):
        e2 = e.reshape(_NW, _EPW)
        fill = jnp.full((_NW, _EPW_PAD - _EPW), pad_val, jnp.int32)
        return jnp.concatenate([e2, fill], axis=1)

    src_lit = _pad(edge_lit, 0).reshape(_NW, _NCHUNK, _K)
    src_cls = _pad(edge_cls, 0).reshape(_NW, _NCHUNK, _K)
    dst_cls = _pad(edge_cls, _N_CLS).reshape(_NW, _NCHUNK, _K)
    dst_lit = _pad(edge_lit, _N_LIT).reshape(_NW, _NCHUNK, _K)
    zeros128 = jnp.zeros((_N_LIT_PAD, _C), jnp.float32)
    ones128 = jnp.ones((_K, _C), jnp.float32)

    cnt_cls = _cnt_cls_kernel(dst_cls, zeros128, ones128)
    cnt_lit = _cnt_lit_kernel(dst_lit, zeros128, ones128)

    x_all = jnp.concatenate([x_lit, x_cls], axis=0)
    h_all = _mlp2(x_all, enc_W1, enc_b1, enc_W2, enc_b2, blk=1000)
    h_lit, h_cls = h_all[:_N_LIT], h_all[_N_LIT:]

    n_layers = lit_W.shape[0]
    cls_W2 = cls_W.reshape(n_layers, 2, _C, _C)
    lit_W2 = lit_W.reshape(n_layers, 2, _C, _C)

    for l in range(n_layers):
        p_cls = _seg_to_cls(h_lit, src_lit, dst_cls, zeros128)
        h_cls = _layer_update(h_cls, p_cls, cnt_cls, cls_W2[l], cls_b[l], blk=1000)
        p_lit = _seg_to_lit(h_cls, src_cls, dst_lit, zeros128)
        h_lit = _layer_update(h_lit, p_lit, cnt_lit, lit_W2[l], lit_b[l], blk=1000)

    hv = h_lit.reshape(_N_CLS, 2 * _C)
    w2p = jnp.zeros((2 * _C, _C), jnp.float32).at[:, :_OUT_DIM].set(out_W2)
    b2p = jnp.zeros((_C,), jnp.float32).at[:_OUT_DIM].set(out_b2)
    y = _mlp2(hv, out_W1, out_b1, w2p, b2p, blk=1000)
    return y[:, :_OUT_DIM]


# speedup vs baseline: 2.5755x; 1.2783x over previous
import functools

import jax
import jax.numpy as jnp
from jax import lax
from jax.experimental import pallas as pl
from jax.experimental.pallas import tpu as pltpu
from jax.experimental.pallas import tpu_sc as plsc

_N_LIT = 10000
_N_CLS = 5000
_E = 320000
_C = 128
_OUT_DIM = 2

_NW = 32
_EPW = _E // _NW
_K = 80
_EPW_PAD = 10000
_NCHUNK = _EPW_PAD // _K
_N_CLS_PAD = 5120
_N_LIT_PAD = 10112

_MESH = plsc.VectorSubcoreMesh(core_axis_name="c", subcore_axis_name="s")


def _make_seg_sum(n_pad):
    stripe = n_pad // 16
    npairs = _NCHUNK // 2

    @functools.partial(
        pl.kernel,
        out_type=jax.ShapeDtypeStruct((2, n_pad, _C), jnp.float32),
        mesh=_MESH,
        scratch_types=[
            pltpu.VMEM((_EPW_PAD,), jnp.int32),
            pltpu.VMEM((_NCHUNK, _K), jnp.int32),
            pltpu.VMEM((_K, _C), jnp.float32),
            pltpu.VMEM((_K, _C), jnp.float32),
            pltpu.MemorySpace.VMEM_SHARED((n_pad, _C), jnp.float32),
            pltpu.SemaphoreType.DMA,
            pltpu.SemaphoreType.DMA,
        ],
    )
    def seg_sum(table, esrc, edst, zeros, out,
                idx_s, idx_d, rows_a, rows_b, acc, sem_a, sem_b):
        c = lax.axis_index("c")
        s = lax.axis_index("s")
        wid = s * 2 + c
        pltpu.sync_copy(zeros.at[pl.ds(s * stripe, stripe)],
                        acc.at[pl.ds(s * stripe, stripe)])
        pltpu.sync_copy(esrc.at[pl.ds(wid * _EPW_PAD, _EPW_PAD)], idx_s)
        pltpu.sync_copy(edst.at[wid], idx_d)
        plsc.subcore_barrier()

        pltpu.async_copy(table.at[idx_s.at[pl.ds(0, _K)]], rows_a, sem_a)

        def body(g, carry):
            ca = 2 * g
            pltpu.make_async_copy(
                table.at[idx_s.at[pl.ds(ca * _K, _K)]], rows_a, sem_a).wait()
            pltpu.async_copy(
                table.at[idx_s.at[pl.ds((ca + 1) * _K, _K)]], rows_b, sem_b)
            pltpu.sync_copy(rows_a, acc.at[idx_d.at[ca]], add=True)
            pltpu.make_async_copy(
                table.at[idx_s.at[pl.ds((ca + 1) * _K, _K)]], rows_b, sem_b).wait()

            @pl.when(g < npairs - 1)
            def _():
                pltpu.async_copy(
                    table.at[idx_s.at[pl.ds((ca + 2) * _K, _K)]], rows_a, sem_a)

            pltpu.sync_copy(rows_b, acc.at[idx_d.at[ca + 1]], add=True)
            return carry

        lax.fori_loop(0, npairs, body, 0)
        if _NCHUNK % 2:
            j = _NCHUNK - 1
            pltpu.async_copy(
                table.at[idx_s.at[pl.ds(j * _K, _K)]], rows_a, sem_a).wait()
            pltpu.sync_copy(rows_a, acc.at[idx_d.at[j]], add=True)
        plsc.subcore_barrier()
        pltpu.sync_copy(acc.at[pl.ds(s * stripe, stripe)],
                        out.at[c, pl.ds(s * stripe, stripe)])

    return seg_sum


_seg_to_cls = _make_seg_sum(_N_CLS_PAD)
_seg_to_lit = _make_seg_sum(_N_LIT_PAD)

def _make_count(n_pad):
    stripe = n_pad // 16

    @functools.partial(
        pl.kernel,
        out_type=jax.ShapeDtypeStruct((2, n_pad, _C), jnp.float32),
        mesh=_MESH,
        scratch_types=[
            pltpu.VMEM((_NCHUNK, _K), jnp.int32),
            pltpu.VMEM((_K, _C), jnp.float32),
            pltpu.MemorySpace.VMEM_SHARED((n_pad, _C), jnp.float32),
        ],
    )
    def count(edst, zeros, ones, out, idx_d, ones_v, acc):
        c = lax.axis_index("c")
        s = lax.axis_index("s")
        wid = s * 2 + c
        pltpu.sync_copy(zeros.at[pl.ds(s * stripe, stripe)],
                        acc.at[pl.ds(s * stripe, stripe)])
        pltpu.sync_copy(ones, ones_v)
        pltpu.sync_copy(edst.at[wid], idx_d)
        plsc.subcore_barrier()

        def body(j, carry):
            pltpu.sync_copy(ones_v, acc.at[idx_d.at[j]], add=True)
            return carry

        lax.fori_loop(0, _NCHUNK, body, 0)
        plsc.subcore_barrier()
        pltpu.sync_copy(acc.at[pl.ds(s * stripe, stripe)],
                        out.at[c, pl.ds(s * stripe, stripe)])

    return count


_cnt_cls_kernel = _make_count(_N_CLS_PAD)
_cnt_lit_kernel = _make_count(_N_LIT_PAD)


def _mlp2(x, w1, b1, w2, b2, blk):
    n, d1 = x.shape
    dh = w1.shape[1]
    do = w2.shape[1]

    def body(x_ref, w1_ref, b1_ref, w2_ref, b2_ref, o_ref):
        z = jnp.dot(x_ref[...], w1_ref[...],
                    preferred_element_type=jnp.float32) + b1_ref[...]
        h = z * jax.nn.sigmoid(z)
        o_ref[...] = jnp.dot(h, w2_ref[...],
                             preferred_element_type=jnp.float32) + b2_ref[...]

    return pl.pallas_call(
        body,
        grid=(n // blk,),
        in_specs=[
            pl.BlockSpec((blk, d1), lambda i: (i, 0)),
            pl.BlockSpec((d1, dh), lambda i: (0, 0)),
            pl.BlockSpec((1, dh), lambda i: (0, 0)),
            pl.BlockSpec((dh, do), lambda i: (0, 0)),
            pl.BlockSpec((1, do), lambda i: (0, 0)),
        ],
        out_specs=pl.BlockSpec((blk, do), lambda i: (i, 0)),
        out_shape=jax.ShapeDtypeStruct((n, do), jnp.float32),
    )(x, w1, b1.reshape(1, dh), w2, b2.reshape(1, do))


def _layer_update(h, partials, cnts, w, b, blk):
    n = h.shape[0]

    def body(h_ref, p_ref, c_ref, w_ref, b_ref, o_ref):
        hx = h_ref[...]
        cnt = c_ref[0] + c_ref[1]
        agg = (p_ref[0] + p_ref[1]) / jnp.maximum(cnt, 1.0)
        z = (jnp.dot(hx, w_ref[0], preferred_element_type=jnp.float32)
             + jnp.dot(agg, w_ref[1], preferred_element_type=jnp.float32)
             + b_ref[...])
        o_ref[...] = hx + z * jax.nn.sigmoid(z)

    return pl.pallas_call(
        body,
        grid=(n // blk,),
        in_specs=[
            pl.BlockSpec((blk, _C), lambda i: (i, 0)),
            pl.BlockSpec((2, blk, _C), lambda i: (0, i, 0)),
            pl.BlockSpec((2, blk, _C), lambda i: (0, i, 0)),
            pl.BlockSpec((2, _C, _C), lambda i: (0, 0, 0)),
            pl.BlockSpec((1, _C), lambda i: (0, 0)),
        ],
        out_specs=pl.BlockSpec((blk, _C), lambda i: (i, 0)),
        out_shape=jax.ShapeDtypeStruct((n, _C), jnp.float32),
    )(h, partials, cnts, w, b.reshape(1, _C))


def kernel(x_lit, x_cls, edge_lit, edge_cls, enc_W1, enc_b1, enc_W2, enc_b2,
           lit_W, lit_b, cls_W, cls_b, out_W1, out_b1, out_W2, out_b2):
    def _pad(e, pad_val):
        e2 = e.reshape(_NW, _EPW)
        fill = jnp.full((_NW, _EPW_PAD - _EPW), pad_val, jnp.int32)
        return jnp.concatenate([e2, fill], axis=1)

    src_lit = _pad(edge_lit, 0).reshape(-1)
    src_cls = _pad(edge_cls, 0).reshape(-1)
    dst_cls = _pad(edge_cls, _N_CLS).reshape(_NW, _NCHUNK, _K)
    dst_lit = _pad(edge_lit, _N_LIT).reshape(_NW, _NCHUNK, _K)
    zeros128 = jnp.zeros((_N_LIT_PAD, _C), jnp.float32)
    ones128 = jnp.ones((_K, _C), jnp.float32)

    cnt_cls = _cnt_cls_kernel(dst_cls, zeros128, ones128)
    cnt_lit = _cnt_lit_kernel(dst_lit, zeros128, ones128)

    x_all = jnp.concatenate([x_lit, x_cls], axis=0)
    h_all = _mlp2(x_all, enc_W1, enc_b1, enc_W2, enc_b2, blk=1000)
    h_lit, h_cls = h_all[:_N_LIT], h_all[_N_LIT:]

    n_layers = lit_W.shape[0]
    cls_W2 = cls_W.reshape(n_layers, 2, _C, _C)
    lit_W2 = lit_W.reshape(n_layers, 2, _C, _C)

    for l in range(n_layers):
        p_cls = _seg_to_cls(h_lit, src_lit, dst_cls, zeros128)
        h_cls = _layer_update(h_cls, p_cls, cnt_cls, cls_W2[l], cls_b[l], blk=1000)
        p_lit = _seg_to_lit(h_cls, src_cls, dst_lit, zeros128)
        h_lit = _layer_update(h_lit, p_lit, cnt_lit, lit_W2[l], lit_b[l], blk=1000)

    hv = h_lit.reshape(_N_CLS, 2 * _C)
    w2p = jnp.zeros((2 * _C, _C), jnp.float32).at[:, :_OUT_DIM].set(out_W2)
    b2p = jnp.zeros((_C,), jnp.float32).at[:_OUT_DIM].set(out_b2)
    y = _mlp2(hv, out_W1, out_b1, w2p, b2p, blk=1000)
    return y[:, :_OUT_DIM]


# speedup vs baseline: 3.2636x; 1.2672x over previous
import functools

import jax
import jax.numpy as jnp
from jax import lax
from jax.experimental import pallas as pl
from jax.experimental.pallas import tpu as pltpu
from jax.experimental.pallas import tpu_sc as plsc

_N_LIT = 10000
_N_CLS = 5000
_E = 320000
_C = 128
_OUT_DIM = 2

_NW = 32
_EPW = _E // _NW
_K = 80
_EPW_PAD = 10000
_NCHUNK = _EPW_PAD // _K
_N_CLS_PAD = 5120
_N_LIT_PAD = 10112

_MESH = plsc.VectorSubcoreMesh(core_axis_name="c", subcore_axis_name="s")


def _make_seg_sum(n_pad):
    stripe = n_pad // 16
    npairs = _NCHUNK // 2

    @functools.partial(
        pl.kernel,
        out_type=jax.ShapeDtypeStruct((2, n_pad, _C), jnp.float32),
        mesh=_MESH,
        scratch_types=[
            pltpu.VMEM((_EPW_PAD,), jnp.int32),
            pltpu.VMEM((_NCHUNK, _K), jnp.int32),
            pltpu.VMEM((_K, _C), jnp.float32),
            pltpu.VMEM((_K, _C), jnp.float32),
            pltpu.MemorySpace.VMEM_SHARED((n_pad, _C), jnp.float32),
            pltpu.SemaphoreType.DMA,
            pltpu.SemaphoreType.DMA,
            pltpu.SemaphoreType.DMA,
            pltpu.SemaphoreType.DMA,
        ],
    )
    def seg_sum(table, esrc, edst, zeros, out,
                idx_s, idx_d, rows_a, rows_b, acc, sem_a, sem_b, sem_c, sem_d):
        c = lax.axis_index("c")
        s = lax.axis_index("s")
        wid = s * 2 + c
        pltpu.sync_copy(zeros.at[pl.ds(s * stripe, stripe)],
                        acc.at[pl.ds(s * stripe, stripe)])
        pltpu.sync_copy(esrc.at[pl.ds(wid * _EPW_PAD, _EPW_PAD)], idx_s)
        pltpu.sync_copy(edst.at[wid], idx_d)
        plsc.subcore_barrier()

        KH = _K // 2
        bufs = [rows_a.at[pl.ds(0, KH)], rows_a.at[pl.ds(KH, KH)],
                rows_b.at[pl.ds(0, KH)], rows_b.at[pl.ds(KH, KH)]]
        sems = [sem_a, sem_b, sem_c, sem_d]

        def fire(q, b):
            pltpu.async_copy(table.at[idx_s.at[pl.ds(q * KH, KH)]],
                             bufs[b], sems[b])

        def wait(q, b):
            pltpu.make_async_copy(table.at[idx_s.at[pl.ds(q * KH, KH)]],
                                  bufs[b], sems[b]).wait()

        for b in range(4):
            fire(b, b)

        def body(g, carry):
            q0 = 4 * g
            wait(q0, 0)
            wait(q0 + 1, 1)
            pltpu.sync_copy(rows_a, acc.at[idx_d.at[2 * g]], add=True)
            for b in range(2):
                @pl.when(q0 + 4 + b < 2 * _NCHUNK)
                def _(b=b):
                    fire(q0 + 4 + b, b)
            wait(q0 + 2, 2)
            wait(q0 + 3, 3)
            pltpu.sync_copy(rows_b, acc.at[idx_d.at[2 * g + 1]], add=True)
            for b in range(2, 4):
                @pl.when(q0 + 4 + b < 2 * _NCHUNK)
                def _(b=b):
                    fire(q0 + 4 + b, b)
            return carry

        lax.fori_loop(0, _NCHUNK // 2, body, 0)
        if _NCHUNK % 2:
            q0 = 2 * (_NCHUNK - 1)
            wait(q0, 0)
            wait(q0 + 1, 1)
            pltpu.sync_copy(rows_a, acc.at[idx_d.at[_NCHUNK - 1]], add=True)
        plsc.subcore_barrier()
        pltpu.sync_copy(acc.at[pl.ds(s * stripe, stripe)],
                        out.at[c, pl.ds(s * stripe, stripe)])

    return seg_sum


_seg_to_cls = _make_seg_sum(_N_CLS_PAD)
_seg_to_lit = _make_seg_sum(_N_LIT_PAD)

def _make_count(n_pad):
    stripe = n_pad // 16

    @functools.partial(
        pl.kernel,
        out_type=jax.ShapeDtypeStruct((2, n_pad, _C), jnp.float32),
        mesh=_MESH,
        scratch_types=[
            pltpu.VMEM((_NCHUNK, _K), jnp.int32),
            pltpu.VMEM((_K, _C), jnp.float32),
            pltpu.MemorySpace.VMEM_SHARED((n_pad, _C), jnp.float32),
        ],
    )
    def count(edst, zeros, ones, out, idx_d, ones_v, acc):
        c = lax.axis_index("c")
        s = lax.axis_index("s")
        wid = s * 2 + c
        pltpu.sync_copy(zeros.at[pl.ds(s * stripe, stripe)],
                        acc.at[pl.ds(s * stripe, stripe)])
        pltpu.sync_copy(ones, ones_v)
        pltpu.sync_copy(edst.at[wid], idx_d)
        plsc.subcore_barrier()

        def body(j, carry):
            pltpu.sync_copy(ones_v, acc.at[idx_d.at[j]], add=True)
            return carry

        lax.fori_loop(0, _NCHUNK, body, 0)
        plsc.subcore_barrier()
        pltpu.sync_copy(acc.at[pl.ds(s * stripe, stripe)],
                        out.at[c, pl.ds(s * stripe, stripe)])

    return count


_cnt_cls_kernel = _make_count(_N_CLS_PAD)
_cnt_lit_kernel = _make_count(_N_LIT_PAD)


def _mlp2(x, w1, b1, w2, b2, blk):
    n, d1 = x.shape
    dh = w1.shape[1]
    do = w2.shape[1]

    def body(x_ref, w1_ref, b1_ref, w2_ref, b2_ref, o_ref):
        z = jnp.dot(x_ref[...], w1_ref[...],
                    preferred_element_type=jnp.float32) + b1_ref[...]
        h = z * jax.nn.sigmoid(z)
        o_ref[...] = jnp.dot(h, w2_ref[...],
                             preferred_element_type=jnp.float32) + b2_ref[...]

    return pl.pallas_call(
        body,
        grid=(n // blk,),
        in_specs=[
            pl.BlockSpec((blk, d1), lambda i: (i, 0)),
            pl.BlockSpec((d1, dh), lambda i: (0, 0)),
            pl.BlockSpec((1, dh), lambda i: (0, 0)),
            pl.BlockSpec((dh, do), lambda i: (0, 0)),
            pl.BlockSpec((1, do), lambda i: (0, 0)),
        ],
        out_specs=pl.BlockSpec((blk, do), lambda i: (i, 0)),
        out_shape=jax.ShapeDtypeStruct((n, do), jnp.float32),
    )(x, w1, b1.reshape(1, dh), w2, b2.reshape(1, do))


def _layer_update(h, partials, cnts, w, b, blk):
    n = h.shape[0]

    def body(h_ref, p_ref, c_ref, w_ref, b_ref, o_ref):
        hx = h_ref[...]
        cnt = c_ref[0] + c_ref[1]
        agg = (p_ref[0] + p_ref[1]) / jnp.maximum(cnt, 1.0)
        z = (jnp.dot(hx, w_ref[0], preferred_element_type=jnp.float32)
             + jnp.dot(agg, w_ref[1], preferred_element_type=jnp.float32)
             + b_ref[...])
        o_ref[...] = hx + z * jax.nn.sigmoid(z)

    return pl.pallas_call(
        body,
        grid=(n // blk,),
        in_specs=[
            pl.BlockSpec((blk, _C), lambda i: (i, 0)),
            pl.BlockSpec((2, blk, _C), lambda i: (0, i, 0)),
            pl.BlockSpec((2, blk, _C), lambda i: (0, i, 0)),
            pl.BlockSpec((2, _C, _C), lambda i: (0, 0, 0)),
            pl.BlockSpec((1, _C), lambda i: (0, 0)),
        ],
        out_specs=pl.BlockSpec((blk, _C), lambda i: (i, 0)),
        out_shape=jax.ShapeDtypeStruct((n, _C), jnp.float32),
    )(h, partials, cnts, w, b.reshape(1, _C))


def kernel(x_lit, x_cls, edge_lit, edge_cls, enc_W1, enc_b1, enc_W2, enc_b2,
           lit_W, lit_b, cls_W, cls_b, out_W1, out_b1, out_W2, out_b2):
    def _pad(e, pad_val):
        e2 = e.reshape(_NW, _EPW)
        fill = jnp.full((_NW, _EPW_PAD - _EPW), pad_val, jnp.int32)
        return jnp.concatenate([e2, fill], axis=1)

    src_lit = _pad(edge_lit, 0).reshape(-1)
    src_cls = _pad(edge_cls, 0).reshape(-1)
    dst_cls = _pad(edge_cls, _N_CLS).reshape(_NW, _NCHUNK, _K)
    dst_lit = _pad(edge_lit, _N_LIT).reshape(_NW, _NCHUNK, _K)
    zeros128 = jnp.zeros((_N_LIT_PAD, _C), jnp.float32)
    ones128 = jnp.ones((_K, _C), jnp.float32)

    cnt_cls = _cnt_cls_kernel(dst_cls, zeros128, ones128)
    cnt_lit = _cnt_lit_kernel(dst_lit, zeros128, ones128)

    x_all = jnp.concatenate([x_lit, x_cls], axis=0)
    h_all = _mlp2(x_all, enc_W1, enc_b1, enc_W2, enc_b2, blk=1000)
    h_lit, h_cls = h_all[:_N_LIT], h_all[_N_LIT:]

    n_layers = lit_W.shape[0]
    cls_W2 = cls_W.reshape(n_layers, 2, _C, _C)
    lit_W2 = lit_W.reshape(n_layers, 2, _C, _C)

    for l in range(n_layers):
        p_cls = _seg_to_cls(h_lit, src_lit, dst_cls, zeros128)
        h_cls = _layer_update(h_cls, p_cls, cnt_cls, cls_W2[l], cls_b[l], blk=1000)
        p_lit = _seg_to_lit(h_cls, src_cls, dst_lit, zeros128)
        h_lit = _layer_update(h_lit, p_lit, cnt_lit, lit_W2[l], lit_b[l], blk=1000)

    hv = h_lit.reshape(_N_CLS, 2 * _C)
    w2p = jnp.zeros((2 * _C, _C), jnp.float32).at[:, :_OUT_DIM].set(out_W2)
    b2p = jnp.zeros((_C,), jnp.float32).at[:_OUT_DIM].set(out_b2)
    y = _mlp2(hv, out_W1, out_b1, w2p, b2p, blk=1000)
    return y[:, :_OUT_DIM]


# speedup vs baseline: 3.2759x; 1.0038x over previous
import functools

import jax
import jax.numpy as jnp
from jax import lax
from jax.experimental import pallas as pl
from jax.experimental.pallas import tpu as pltpu
from jax.experimental.pallas import tpu_sc as plsc

_N_LIT = 10000
_N_CLS = 5000
_E = 320000
_C = 128
_OUT_DIM = 2

_NW = 32
_EPW = _E // _NW
_K = 80
_EPW_PAD = 10000
_NCHUNK = _EPW_PAD // _K
_N_CLS_PAD = 5120
_N_LIT_PAD = 10112

_MESH = plsc.VectorSubcoreMesh(core_axis_name="c", subcore_axis_name="s")


def _make_seg_sum(n_pad):
    stripe = n_pad // 16
    npairs = _NCHUNK // 2

    @functools.partial(
        pl.kernel,
        out_type=jax.ShapeDtypeStruct((2, n_pad, _C), jnp.float32),
        mesh=_MESH,
        scratch_types=[
            pltpu.VMEM((_EPW_PAD,), jnp.int32),
            pltpu.VMEM((_NCHUNK, _K), jnp.int32),
            pltpu.VMEM((_K, _C), jnp.float32),
            pltpu.VMEM((_K, _C), jnp.float32),
            pltpu.MemorySpace.VMEM_SHARED((n_pad, _C), jnp.float32),
            pltpu.SemaphoreType.DMA,
            pltpu.SemaphoreType.DMA,
            pltpu.SemaphoreType.DMA,
            pltpu.SemaphoreType.DMA,
        ],
    )
    def seg_sum(table, esrc, edst, zeros, out,
                idx_s, idx_d, rows_a, rows_b, acc, sem_a, sem_b, sem_c, sem_d):
        c = lax.axis_index("c")
        s = lax.axis_index("s")
        wid = s * 2 + c
        pltpu.sync_copy(zeros.at[pl.ds(s * stripe, stripe)],
                        acc.at[pl.ds(s * stripe, stripe)])
        pltpu.sync_copy(esrc.at[pl.ds(wid * _EPW_PAD, _EPW_PAD)], idx_s)
        pltpu.sync_copy(edst.at[wid], idx_d)
        plsc.subcore_barrier()

        KH = _K // 2
        bufs = [rows_a.at[pl.ds(0, KH)], rows_a.at[pl.ds(KH, KH)],
                rows_b.at[pl.ds(0, KH)], rows_b.at[pl.ds(KH, KH)]]
        sems = [sem_a, sem_b, sem_c, sem_d]

        def fire(q, b):
            pltpu.async_copy(table.at[idx_s.at[pl.ds(q * KH, KH)]],
                             bufs[b], sems[b])

        def wait(q, b):
            pltpu.make_async_copy(table.at[idx_s.at[pl.ds(q * KH, KH)]],
                                  bufs[b], sems[b]).wait()

        for b in range(4):
            fire(b, b)

        def body(g, carry):
            q0 = 4 * g
            wait(q0, 0)
            wait(q0 + 1, 1)
            pltpu.sync_copy(rows_a, acc.at[idx_d.at[2 * g]], add=True)
            for b in range(2):
                @pl.when(q0 + 4 + b < 2 * _NCHUNK)
                def _(b=b):
                    fire(q0 + 4 + b, b)
            wait(q0 + 2, 2)
            wait(q0 + 3, 3)
            pltpu.sync_copy(rows_b, acc.at[idx_d.at[2 * g + 1]], add=True)
            for b in range(2, 4):
                @pl.when(q0 + 4 + b < 2 * _NCHUNK)
                def _(b=b):
                    fire(q0 + 4 + b, b)
            return carry

        lax.fori_loop(0, _NCHUNK // 2, body, 0)
        if _NCHUNK % 2:
            q0 = 2 * (_NCHUNK - 1)
            wait(q0, 0)
            wait(q0 + 1, 1)
            pltpu.sync_copy(rows_a, acc.at[idx_d.at[_NCHUNK - 1]], add=True)
        plsc.subcore_barrier()
        pltpu.sync_copy(acc.at[pl.ds(s * stripe, stripe)],
                        out.at[c, pl.ds(s * stripe, stripe)])

    return seg_sum


_seg_to_cls = _make_seg_sum(_N_CLS_PAD)
_seg_to_lit = _make_seg_sum(_N_LIT_PAD)

def _make_count(n_pad):
    stripe = n_pad // 16

    @functools.partial(
        pl.kernel,
        out_type=jax.ShapeDtypeStruct((2, n_pad, _C), jnp.float32),
        mesh=_MESH,
        scratch_types=[
            pltpu.VMEM((_NCHUNK, _K), jnp.int32),
            pltpu.VMEM((_K, _C), jnp.float32),
            pltpu.MemorySpace.VMEM_SHARED((n_pad, _C), jnp.float32),
            pltpu.SemaphoreType.DMA,
        ],
    )
    def count(edst, zeros, ones, out, idx_d, ones_v, acc, sem):
        c = lax.axis_index("c")
        s = lax.axis_index("s")
        wid = s * 2 + c
        pltpu.sync_copy(zeros.at[pl.ds(s * stripe, stripe)],
                        acc.at[pl.ds(s * stripe, stripe)])
        pltpu.sync_copy(ones, ones_v)
        pltpu.sync_copy(edst.at[wid], idx_d)
        plsc.subcore_barrier()

        def body(j, carry):
            pltpu.async_copy(ones_v, acc.at[idx_d.at[j]], sem, add=True)
            return carry

        lax.fori_loop(0, _NCHUNK, body, 0)

        def drain(j, carry):
            pltpu.make_async_copy(ones_v, acc.at[idx_d.at[0]], sem).wait()
            return carry

        lax.fori_loop(0, _NCHUNK, drain, 0)
        plsc.subcore_barrier()
        pltpu.sync_copy(acc.at[pl.ds(s * stripe, stripe)],
                        out.at[c, pl.ds(s * stripe, stripe)])

    return count


_cnt_cls_kernel = _make_count(_N_CLS_PAD)
_cnt_lit_kernel = _make_count(_N_LIT_PAD)


def _mlp2(x, w1, b1, w2, b2, blk):
    n, d1 = x.shape
    dh = w1.shape[1]
    do = w2.shape[1]

    def body(x_ref, w1_ref, b1_ref, w2_ref, b2_ref, o_ref):
        z = jnp.dot(x_ref[...], w1_ref[...],
                    preferred_element_type=jnp.float32) + b1_ref[...]
        h = z * jax.nn.sigmoid(z)
        o_ref[...] = jnp.dot(h, w2_ref[...],
                             preferred_element_type=jnp.float32) + b2_ref[...]

    return pl.pallas_call(
        body,
        grid=(n // blk,),
        in_specs=[
            pl.BlockSpec((blk, d1), lambda i: (i, 0)),
            pl.BlockSpec((d1, dh), lambda i: (0, 0)),
            pl.BlockSpec((1, dh), lambda i: (0, 0)),
            pl.BlockSpec((dh, do), lambda i: (0, 0)),
            pl.BlockSpec((1, do), lambda i: (0, 0)),
        ],
        out_specs=pl.BlockSpec((blk, do), lambda i: (i, 0)),
        out_shape=jax.ShapeDtypeStruct((n, do), jnp.float32),
    )(x, w1, b1.reshape(1, dh), w2, b2.reshape(1, do))


def _layer_update(h, partials, cnts, w, b, blk):
    n = h.shape[0]

    def body(h_ref, p_ref, c_ref, w_ref, b_ref, o_ref):
        hx = h_ref[...]
        cnt = c_ref[0] + c_ref[1]
        agg = (p_ref[0] + p_ref[1]) / jnp.maximum(cnt, 1.0)
        z = (jnp.dot(hx, w_ref[0], preferred_element_type=jnp.float32)
             + jnp.dot(agg, w_ref[1], preferred_element_type=jnp.float32)
             + b_ref[...])
        o_ref[...] = hx + z * jax.nn.sigmoid(z)

    return pl.pallas_call(
        body,
        grid=(n // blk,),
        in_specs=[
            pl.BlockSpec((blk, _C), lambda i: (i, 0)),
            pl.BlockSpec((2, blk, _C), lambda i: (0, i, 0)),
            pl.BlockSpec((2, blk, _C), lambda i: (0, i, 0)),
            pl.BlockSpec((2, _C, _C), lambda i: (0, 0, 0)),
            pl.BlockSpec((1, _C), lambda i: (0, 0)),
        ],
        out_specs=pl.BlockSpec((blk, _C), lambda i: (i, 0)),
        out_shape=jax.ShapeDtypeStruct((n, _C), jnp.float32),
    )(h, partials, cnts, w, b.reshape(1, _C))


def kernel(x_lit, x_cls, edge_lit, edge_cls, enc_W1, enc_b1, enc_W2, enc_b2,
           lit_W, lit_b, cls_W, cls_b, out_W1, out_b1, out_W2, out_b2):
    def _pad(e, pad_val):
        e2 = e.reshape(_NW, _EPW)
        fill = jnp.full((_NW, _EPW_PAD - _EPW), pad_val, jnp.int32)
        return jnp.concatenate([e2, fill], axis=1)

    src_lit = _pad(edge_lit, 0).reshape(-1)
    src_cls = _pad(edge_cls, 0).reshape(-1)
    dst_cls = _pad(edge_cls, _N_CLS).reshape(_NW, _NCHUNK, _K)
    dst_lit = _pad(edge_lit, _N_LIT).reshape(_NW, _NCHUNK, _K)
    zeros128 = jnp.zeros((_N_LIT_PAD, _C), jnp.float32)
    ones128 = jnp.ones((_K, _C), jnp.float32)

    cnt_cls = _cnt_cls_kernel(dst_cls, zeros128, ones128)
    cnt_lit = _cnt_lit_kernel(dst_lit, zeros128, ones128)

    x_all = jnp.concatenate([x_lit, x_cls], axis=0)
    h_all = _mlp2(x_all, enc_W1, enc_b1, enc_W2, enc_b2, blk=1000)
    h_lit, h_cls = h_all[:_N_LIT], h_all[_N_LIT:]

    n_layers = lit_W.shape[0]
    cls_W2 = cls_W.reshape(n_layers, 2, _C, _C)
    lit_W2 = lit_W.reshape(n_layers, 2, _C, _C)

    for l in range(n_layers):
        p_cls = _seg_to_cls(h_lit, src_lit, dst_cls, zeros128)
        h_cls = _layer_update(h_cls, p_cls, cnt_cls, cls_W2[l], cls_b[l], blk=1000)
        p_lit = _seg_to_lit(h_cls, src_cls, dst_lit, zeros128)
        h_lit = _layer_update(h_lit, p_lit, cnt_lit, lit_W2[l], lit_b[l], blk=1000)

    hv = h_lit.reshape(_N_CLS, 2 * _C)
    w2p = jnp.zeros((2 * _C, _C), jnp.float32).at[:, :_OUT_DIM].set(out_W2)
    b2p = jnp.zeros((_C,), jnp.float32).at[:_OUT_DIM].set(out_b2)
    y = _mlp2(hv, out_W1, out_b1, w2p, b2p, blk=1000)
    return y[:, :_OUT_DIM]
